# baseline (speedup 1.0000x reference)
import functools

import jax
import jax.numpy as jnp
from jax import lax
from jax.experimental import pallas as pl
from jax.experimental.pallas import tpu as pltpu
from jax.experimental.pallas import tpu_sc as plsc

N_NODES = 10000
D = 128
N_EDGES = 320000

NC = 2
NS = 16
NW = NC * NS

CHUNK = 128
NCHUNK = 80
EPW = NCHUNK * CHUNK
NPAD = 10240
RPT = NPAD // NS
LANES = 16


def _sc_aggregate(x, idx):
    mesh = plsc.VectorSubcoreMesh(
        core_axis_name="c", subcore_axis_name="s", num_cores=NC,
        num_subcores=NS)

    @functools.partial(
        pl.kernel,
        out_type=(
            jax.ShapeDtypeStruct((NC, NPAD, D), jnp.float32),
            jax.ShapeDtypeStruct((NC, NPAD), jnp.float32),
        ),
        mesh=mesh,
        scratch_types=[
            pltpu.VMEM_SHARED((NPAD, D), jnp.float32),
            pltpu.VMEM_SHARED((NPAD,), jnp.float32),
            pltpu.VMEM((2, CHUNK), jnp.int32),
            pltpu.VMEM((2, CHUNK), jnp.int32),
            pltpu.VMEM((CHUNK, D), jnp.float32),
            pltpu.VMEM((CHUNK, D), jnp.float32),
            pltpu.VMEM((RPT,), jnp.float32),
            pltpu.VMEM((CHUNK,), jnp.float32),
            pltpu.SemaphoreType.DMA,
            pltpu.SemaphoreType.DMA,
            pltpu.SemaphoreType.DMA,
            pltpu.SemaphoreType.DMA,
        ],
    )
    def k(x_hbm, idx_hbm, pagg_hbm, pcnt_hbm,
          agg_sh, cnt_sh, idx0_v, idx1_v, rows0_v, rows1_v, zc_v, ones_v,
          rsem0, rsem1, isem0, isem1):
        c = lax.axis_index("c")
        s = lax.axis_index("s")
        wid = c * NS + s
        base = s * RPT

        rows = (rows0_v, rows1_v)
        idxs = (idx0_v, idx1_v)
        rsems = (rsem0, rsem1)
        isems = (isem0, isem1)

        def zero_row(i, _):
            for kk in range(D // LANES):
                rows0_v[i, pl.ds(kk * LANES, LANES)] = jnp.zeros(
                    (LANES,), jnp.float32)
            return ()
        lax.fori_loop(0, CHUNK, zero_row, ())
        for kk in range(RPT // LANES):
            zc_v[pl.ds(kk * LANES, LANES)] = jnp.zeros((LANES,), jnp.float32)
        for kk in range(CHUNK // LANES):
            ones_v[pl.ds(kk * LANES, LANES)] = jnp.ones((LANES,), jnp.float32)

        for kk in range(RPT // CHUNK):
            pltpu.sync_copy(rows0_v,
                            agg_sh.at[pl.ds(base + kk * CHUNK, CHUNK), :])
        pltpu.sync_copy(zc_v, cnt_sh.at[pl.ds(base, RPT)])

        pltpu.async_copy(idx_hbm.at[wid, 0], idx0_v, isem0)
        pltpu.async_copy(idx_hbm.at[wid, 1], idx1_v, isem1)
        pltpu.make_async_copy(idx_hbm.at[wid, 0], idx0_v, isem0).wait()
        pltpu.async_copy(x_hbm.at[idx0_v.at[0]], rows0_v, rsem0)

        plsc.subcore_barrier()

        def step(i, _):
            for bb in range(2):
                cur = 2 * i + bb
                pltpu.make_async_copy(
                    x_hbm.at[idxs[bb].at[0]], rows[bb], rsems[bb]).wait()

                @pl.when(cur + 1 < NCHUNK)
                def _():
                    pltpu.make_async_copy(
                        idx_hbm.at[wid, cur + 1], idxs[1 - bb],
                        isems[1 - bb]).wait()
                    pltpu.async_copy(x_hbm.at[idxs[1 - bb].at[0]],
                                     rows[1 - bb], rsems[1 - bb])
                pltpu.sync_copy(rows[bb], agg_sh.at[idxs[bb].at[1]], add=True)
                pltpu.sync_copy(ones_v, cnt_sh.at[idxs[bb].at[1]], add=True)

                @pl.when(cur + 2 < NCHUNK)
                def _():
                    pltpu.async_copy(idx_hbm.at[wid, cur + 2], idxs[bb],
                                     isems[bb])
            return ()
        lax.fori_loop(0, NCHUNK // 2, step, ())

        plsc.subcore_barrier()

        pltpu.sync_copy(agg_sh.at[pl.ds(base, RPT), :],
                        pagg_hbm.at[c, pl.ds(base, RPT), :])
        pltpu.sync_copy(cnt_sh.at[pl.ds(base, RPT)],
                        pcnt_hbm.at[c, pl.ds(base, RPT)])

    return k(x, idx)


BLK = 1024


def _tc_body(pa_ref, pc_ref, w_ref, b_ref, o_ref):
    a = pa_ref[0] + pa_ref[1]
    cnt = pc_ref[0] + pc_ref[1] + 1e-6
    a = a / cnt[:, None]
    o_ref[...] = lax.dot_general(
        a, w_ref[...], (((1,), (1,)), ((), ())),
        preferred_element_type=jnp.float32) + b_ref[...]


def _tc_linear(pagg, pcnt, W, b):
    return pl.pallas_call(
        _tc_body,
        grid=(NPAD // BLK,),
        in_specs=[
            pl.BlockSpec((NC, BLK, D), lambda i: (0, i, 0)),
            pl.BlockSpec((NC, BLK), lambda i: (0, i)),
            pl.BlockSpec((D, D), lambda i: (0, 0)),
            pl.BlockSpec((1, D), lambda i: (0, 0)),
        ],
        out_specs=pl.BlockSpec((BLK, D), lambda i: (i, 0)),
        out_shape=jax.ShapeDtypeStruct((NPAD, D), jnp.float32),
    )(pagg, pcnt, W, b)


def kernel(x, edge_index, W, b):
    src = edge_index[0]
    tgt = edge_index[1]
    pad = NW * EPW - N_EDGES
    src_p = jnp.concatenate(
        [src, jnp.full((pad,), N_NODES, dtype=jnp.int32)])
    tgt_p = jnp.concatenate([tgt, jnp.zeros((pad,), dtype=jnp.int32)])
    idx = jnp.stack(
        [tgt_p.reshape(NW, NCHUNK, CHUNK), src_p.reshape(NW, NCHUNK, CHUNK)],
        axis=2)

    pagg, pcnt = _sc_aggregate(x, idx)
    out = _tc_linear(pagg, pcnt, W, b.reshape(1, D))
    return out[:N_NODES]


# speedup vs baseline: 1.0002x; 1.0002x over previous
import functools

import jax
import jax.numpy as jnp
from jax import lax
from jax.experimental import pallas as pl
from jax.experimental.pallas import tpu as pltpu
from jax.experimental.pallas import tpu_sc as plsc

N_NODES = 10000
D = 128
N_EDGES = 320000

NC = 2
NS = 16
NW = NC * NS

CHUNK = 128
NCHUNK = 80
EPW = NCHUNK * CHUNK
NPAD = 10112
RPT = NPAD // NS
LANES = 16


def _sc_aggregate(x, srcs, tgts):
    mesh = plsc.VectorSubcoreMesh(
        core_axis_name="c", subcore_axis_name="s", num_cores=NC,
        num_subcores=NS)

    @functools.partial(
        pl.kernel,
        out_type=(
            jax.ShapeDtypeStruct((NC * NPAD, D), jnp.float32),
            jax.ShapeDtypeStruct((NC * NPAD,), jnp.float32),
        ),
        mesh=mesh,
        scratch_types=[
            pltpu.VMEM_SHARED((NPAD, D), jnp.float32),
            pltpu.VMEM_SHARED((NPAD,), jnp.float32),
            pltpu.VMEM((NCHUNK // 2, CHUNK), jnp.int32),
            pltpu.VMEM((NCHUNK // 2, CHUNK), jnp.int32),
            pltpu.VMEM((2 * CHUNK, D), jnp.float32),
            pltpu.VMEM((640,), jnp.float32),
            pltpu.VMEM((CHUNK,), jnp.float32),
            pltpu.SemaphoreType.DMA,
            pltpu.SemaphoreType.DMA,
        ],
    )
    def k(x_hbm, srcs_hbm, tgts_hbm, pagg_hbm, pcnt_hbm,
          agg_sh, cnt_sh, src_v, tgt_v, rows_v, zc_v, ones_v,
          sem0, sem1):
        c = lax.axis_index("c")
        s = lax.axis_index("s")
        wid = c * NS + s
        base = s * RPT

        def slot(bb):
            return rows_v.at[pl.ds(bb * CHUNK, CHUNK), :]
        sems = (sem0, sem1)

        def zero_row(i, _):
            for kk in range(D // LANES):
                rows_v[i, pl.ds(kk * LANES, LANES)] = jnp.zeros(
                    (LANES,), jnp.float32)
            return ()
        lax.fori_loop(0, 2 * CHUNK, zero_row, ())
        for kk in range(640 // LANES):
            zc_v[pl.ds(kk * LANES, LANES)] = jnp.zeros((LANES,), jnp.float32)
        for kk in range(CHUNK // LANES):
            ones_v[pl.ds(kk * LANES, LANES)] = jnp.ones((LANES,), jnp.float32)

        pltpu.sync_copy(rows_v, agg_sh.at[pl.ds(base, 2 * CHUNK), :])
        pltpu.sync_copy(rows_v,
                        agg_sh.at[pl.ds(base + 2 * CHUNK, 2 * CHUNK), :])
        pltpu.sync_copy(rows_v.at[pl.ds(0, RPT - 4 * CHUNK), :],
                        agg_sh.at[pl.ds(base + 4 * CHUNK, RPT - 4 * CHUNK), :])
        @pl.when(s < NS - 1)
        def _():
            pltpu.sync_copy(zc_v, cnt_sh.at[pl.ds(s * 640, 640)])

        @pl.when(s == NS - 1)
        def _():
            pltpu.sync_copy(zc_v.at[pl.ds(0, 512)],
                            cnt_sh.at[pl.ds(15 * 640, 512)])

        PH = NCHUNK // 2
        for p in range(2):
            pltpu.sync_copy(srcs_hbm.at[wid, pl.ds(p * PH, PH)], src_v)
            pltpu.sync_copy(tgts_hbm.at[wid, pl.ds(p * PH, PH)], tgt_v)
            pltpu.async_copy(x_hbm.at[tgt_v.at[0]], slot(0), sem0)
            if p == 0:
                plsc.subcore_barrier()

            def step(i, _):
                for bb in range(2):
                    cur = 2 * i + bb
                    pltpu.make_async_copy(
                        x_hbm.at[tgt_v.at[cur]], slot(bb), sems[bb]).wait()

                    @pl.when(cur + 1 < PH)
                    def _():
                        pltpu.async_copy(x_hbm.at[tgt_v.at[cur + 1]],
                                         slot(1 - bb), sems[1 - bb])
                    pltpu.sync_copy(slot(bb), agg_sh.at[src_v.at[cur]],
                                    add=True)
                    pltpu.sync_copy(ones_v, cnt_sh.at[src_v.at[cur]],
                                    add=True)
                return ()
            lax.fori_loop(0, PH // 2, step, ())

        plsc.subcore_barrier()

        pltpu.sync_copy(agg_sh.at[pl.ds(base, RPT), :],
                        pagg_hbm.at[pl.ds(c * NPAD + base, RPT), :])
        @pl.when(s < NS - 1)
        def _():
            pltpu.sync_copy(
                cnt_sh.at[pl.ds(s * 640, 640)],
                pcnt_hbm.at[pl.ds(c * NPAD + s * 640, 640)])

        @pl.when(s == NS - 1)
        def _():
            pltpu.sync_copy(
                cnt_sh.at[pl.ds(15 * 640, 512)],
                pcnt_hbm.at[pl.ds(c * NPAD + 15 * 640, 512)])

    return k(x, srcs, tgts)


def _tc_body(pa_ref, pc_ref, w_ref, b_ref, o_ref):
    a = jnp.sum(pa_ref[...], axis=0)
    cnt = jnp.sum(pc_ref[...], axis=0) + 1e-6
    a = a / cnt[:, None]
    o_ref[...] = lax.dot_general(
        a, w_ref[...], (((1,), (1,)), ((), ())),
        preferred_element_type=jnp.float32) + b_ref[...]


def _tc_linear(pagg, pcnt, W, b):
    return pl.pallas_call(
        _tc_body,
        out_shape=jax.ShapeDtypeStruct((NPAD, D), jnp.float32),
    )(pagg, pcnt, W, b)


def kernel(x, edge_index, W, b):
    src = edge_index[0]
    tgt = edge_index[1]
    pad = NW * EPW - N_EDGES
    src_p = jnp.concatenate(
        [src, jnp.full((pad,), N_NODES, dtype=jnp.int32)])
    tgt_p = jnp.concatenate([tgt, jnp.zeros((pad,), dtype=jnp.int32)])
    srcs = src_p.reshape(NW, NCHUNK, CHUNK)
    tgts = tgt_p.reshape(NW, NCHUNK, CHUNK)

    pagg, pcnt = _sc_aggregate(x, srcs, tgts)
    pagg = pagg.reshape(NC, NPAD, D)
    pcnt = pcnt.reshape(NC, NPAD)
    out = _tc_linear(pagg, pcnt, W, b.reshape(1, D))
    return out[:N_NODES]
